# baseline (speedup 1.0000x reference)
import functools

import jax
import jax.numpy as jnp
from jax import lax
from jax.experimental import pallas as pl
from jax.experimental.pallas import tpu as pltpu
from jax.experimental.pallas import tpu_sc as plsc

_VOCAB = 1000000
_D = 64
_BATCH = 4096
_SEQ = 200
_NC = 2
_NS = 16
_NW = _NC * _NS
_NPANEL = _VOCAB // 128
_TAIL = _VOCAB - _NPANEL * 128
_PAIR_ROWS = _VOCAB // 2
_SCALE = 8.0

_mesh = plsc.VectorSubcoreMesh(core_axis_name="c", subcore_axis_name="s")


@functools.partial(
    pl.kernel,
    mesh=_mesh,
    out_type=jax.ShapeDtypeStruct((_PAIR_ROWS, 128), jnp.float32),
    scratch_types=(
        [pltpu.VMEM((_D, 128), jnp.float32)] * 2
        + [pltpu.VMEM((_D, 128), jnp.float32)] * 2
        + [pltpu.SemaphoreType.DMA] * 4
    ),
    compiler_params=pltpu.CompilerParams(use_tc_tiling_on_sc=True, needs_layout_passes=False),
)
def _transpose_kernel(tabT_hbm, tail_hbm, out_hbm, p0, p1, o0, o1,
                      gs0, gs1, ss0, ss1):
    wid = lax.axis_index("s") * _NC + lax.axis_index("c")
    pbufs, obufs = (p0, p1), (o0, o1)
    gsems, ssems = (gs0, gs1), (ss0, ss1)

    n_t = (_NPANEL - 1 - wid) // _NW + 1

    row_idx = [lax.iota(jnp.int32, 16) + 16 * k for k in range(4)]

    def fire_in(c, b):
        pltpu.async_copy(tabT_hbm.at[:, pl.ds(c * 128, 128)], pbufs[b], gsems[b])

    def wait_in(c, b):
        pltpu.make_async_copy(
            tabT_hbm.at[:, pl.ds(c * 128, 128)], pbufs[b], gsems[b]).wait()

    def fire_out(c, b):
        pltpu.async_copy(obufs[b], out_hbm.at[pl.ds(c * 64, 64)], ssems[b])

    def wait_out(c, b):
        pltpu.make_async_copy(
            obufs[b], out_hbm.at[pl.ds(c * 64, 64)], ssems[b]).wait()

    def permute(p, o):
        def body(r, carry):
            for par in range(2):
                col = 2 * r + par
                for k in range(4):
                    v = plsc.load_gather(
                        p, [row_idx[k], jnp.full((16,), 0, jnp.int32) + col])
                    o[r, pl.ds(par * 64 + 16 * k, 16)] = v
            return carry
        lax.fori_loop(0, _D, body, 0, unroll=2)

    fire_in(wid, 0)

    def loop(t, carry):
        for b in range(2):
            c = (2 * t + b) * _NW + wid

            @pl.when(c < _NPANEL)
            def _():
                nc = c + _NW

                @pl.when(nc < _NPANEL)
                def _():
                    fire_in(nc, 1 - b)
                wait_in(c, b)

                @pl.when(c >= 2 * _NW)
                def _():
                    wait_out(c - 2 * _NW, b)
                permute(pbufs[b], obufs[b])
                fire_out(c, b)
        return carry

    lax.fori_loop(0, (n_t + 1) // 2, loop, 0)

    @pl.when(n_t == 245)
    def _():
        wait_out(243 * _NW + wid, 1)
        wait_out(244 * _NW + wid, 0)

    @pl.when(n_t == 244)
    def _():
        wait_out(242 * _NW + wid, 0)
        wait_out(243 * _NW + wid, 1)

    @pl.when(wid == _NW - 1)
    def _():
        pltpu.sync_copy(tail_hbm, o0.at[pl.ds(0, _TAIL // 2)])
        pltpu.sync_copy(
            o0.at[pl.ds(0, _TAIL // 2)],
            out_hbm.at[pl.ds(_NPANEL * 64, _TAIL // 2)])


@functools.partial(
    pl.kernel,
    mesh=_mesh,
    out_type=jax.ShapeDtypeStruct((_SEQ, 8, _NW, 8, 128), jnp.float32),
    scratch_types=(
        [pltpu.VMEM((_SEQ, 128), jnp.int32)]
        + [pltpu.VMEM((_SEQ, 128), jnp.int32)]
        + [pltpu.VMEM((128, 128), jnp.float32)] * 2
        + [pltpu.VMEM((8, 8, 128), jnp.float32)] * 2
        + [pltpu.SemaphoreType.DMA] * 4
    ),
    compiler_params=pltpu.CompilerParams(use_tc_tiling_on_sc=True, needs_layout_passes=False),
)
def _gather_kernel(xT_hbm, tab_hbm, out_hbm, idx_v, off_v,
                   g0, g1, o0, o1, gs0, gs1, ss0, ss1):
    wid = lax.axis_index("s") * _NC + lax.axis_index("c")
    gbufs, obufs = (g0, g1), (o0, o1)
    gsems, ssems = (gs0, gs1), (ss0, ss1)

    pltpu.sync_copy(xT_hbm.at[:, pl.ds(wid * 128, 128)], idx_v)

    def prep(s, carry):
        for g in range(8):
            ix = idx_v[s, pl.ds(16 * g, 16)]
            off_v[s, pl.ds(16 * g, 16)] = (ix & 1) << 6
            idx_v[s, pl.ds(16 * g, 16)] = lax.shift_right_logical(ix, 1)
        return carry
    lax.fori_loop(0, _SEQ, prep, 0, unroll=2)

    def fire_in(s, b):
        pltpu.async_copy(tab_hbm.at[idx_v.at[s]], gbufs[b], gsems[b])

    def wait_in(s, b):
        pltpu.make_async_copy(tab_hbm.at[idx_v.at[s]], gbufs[b], gsems[b]).wait()

    def fire_out(s, b):
        pltpu.async_copy(obufs[b], out_hbm.at[s, :, wid], ssems[b])

    def wait_out(s, b):
        pltpu.make_async_copy(obufs[b], out_hbm.at[s, :, wid], ssems[b]).wait()

    bi_idx = [lax.iota(jnp.int32, 16) + 16 * g for g in range(8)]

    def permute(s, g, o):
        def body(d, carry):
            d0 = lax.shift_right_logical(d, 3)
            di = d & 7
            for grp in range(8):
                off = off_v[s, pl.ds(16 * grp, 16)] + d
                v = plsc.load_gather(g, [bi_idx[grp], off])
                o[d0, di, pl.ds(16 * grp, 16)] = v * _SCALE
            return carry
        lax.fori_loop(0, _D, body, 0, unroll=2)

    fire_in(0, 0)

    def loop(t, carry):
        for b in range(2):
            s = 2 * t + b
            ns = s + 1

            @pl.when(ns < _SEQ)
            def _():
                fire_in(ns, 1 - b)
            wait_in(s, b)

            @pl.when(s >= 2)
            def _():
                wait_out(s - 2, b)
            permute(s, gbufs[b], obufs[b])
            fire_out(s, b)
        return carry

    lax.fori_loop(0, _SEQ // 2, loop, 0)
    wait_out(_SEQ - 2, 0)
    wait_out(_SEQ - 1, 1)


def kernel(x, input_embedding_table):
    tail = input_embedding_table[_NPANEL * 128:].reshape(_TAIL // 2, 128)
    tab2 = _transpose_kernel(input_embedding_table.T, tail)
    out5 = _gather_kernel(x.T, tab2)
    return out5.transpose(2, 4, 0, 1, 3).reshape(_BATCH, _SEQ, _D)


# speedup vs baseline: 2.4258x; 2.4258x over previous
import functools

import jax
import jax.numpy as jnp
from jax import lax
from jax.experimental import pallas as pl
from jax.experimental.pallas import tpu as pltpu
from jax.experimental.pallas import tpu_sc as plsc

_VOCAB = 1000000
_D = 64
_BATCH = 4096
_SEQ = 200
_NC = 2
_NS = 16
_NW = _NC * _NS
_NPANEL = _VOCAB // 128
_TAIL = _VOCAB - _NPANEL * 128
_PAIR_ROWS = _VOCAB // 2
_SCALE = 8.0

_mesh = plsc.VectorSubcoreMesh(core_axis_name="c", subcore_axis_name="s")


@functools.partial(
    pl.kernel,
    mesh=_mesh,
    out_type=jax.ShapeDtypeStruct((_PAIR_ROWS, 128), jnp.float32),
    scratch_types=(
        [pltpu.VMEM((_D, 128), jnp.float32)] * 2
        + [pltpu.VMEM((_D, 128), jnp.float32)] * 2
        + [pltpu.SemaphoreType.DMA] * 4
    ),
    compiler_params=pltpu.CompilerParams(use_tc_tiling_on_sc=True, needs_layout_passes=False),
)
def _transpose_kernel(tabT_hbm, tail_hbm, out_hbm, p0, p1, o0, o1,
                      gs0, gs1, ss0, ss1):
    wid = lax.axis_index("s") * _NC + lax.axis_index("c")
    pbufs, obufs = (p0, p1), (o0, o1)
    gsems, ssems = (gs0, gs1), (ss0, ss1)

    n_t = (_NPANEL - 1 - wid) // _NW + 1

    row_idx = [lax.iota(jnp.int32, 16) + 16 * k for k in range(4)]

    def fire_in(c, b):
        pltpu.async_copy(tabT_hbm.at[:, pl.ds(c * 128, 128)], pbufs[b], gsems[b])

    def wait_in(c, b):
        pltpu.make_async_copy(
            tabT_hbm.at[:, pl.ds(c * 128, 128)], pbufs[b], gsems[b]).wait()

    def fire_out(c, b):
        pltpu.async_copy(obufs[b], out_hbm.at[pl.ds(c * 64, 64)], ssems[b])

    def wait_out(c, b):
        pltpu.make_async_copy(
            obufs[b], out_hbm.at[pl.ds(c * 64, 64)], ssems[b]).wait()

    zeros16 = jnp.full((16,), 0, jnp.int32)

    def permute(p, o):
        @plsc.parallel_loop(0, _D, unroll=4)
        def _(r):
            for par in range(2):
                col = 2 * r + par
                for k in range(4):
                    v = plsc.load_gather(p, [row_idx[k], zeros16 + col])
                    o[r, pl.ds(par * 64 + 16 * k, 16)] = v

    fire_in(wid, 0)

    def loop(t, carry):
        for b in range(2):
            c = (2 * t + b) * _NW + wid

            @pl.when(c < _NPANEL)
            def _():
                nc = c + _NW

                @pl.when(nc < _NPANEL)
                def _():
                    fire_in(nc, 1 - b)
                wait_in(c, b)

                @pl.when(c >= 2 * _NW)
                def _():
                    wait_out(c - 2 * _NW, b)
                permute(pbufs[b], obufs[b])
                fire_out(c, b)
        return carry

    lax.fori_loop(0, (n_t + 1) // 2, loop, 0)

    @pl.when(n_t == 245)
    def _():
        wait_out(243 * _NW + wid, 1)
        wait_out(244 * _NW + wid, 0)

    @pl.when(n_t == 244)
    def _():
        wait_out(242 * _NW + wid, 0)
        wait_out(243 * _NW + wid, 1)

    @pl.when(wid == _NW - 1)
    def _():
        pltpu.sync_copy(tail_hbm, o0.at[pl.ds(0, _TAIL // 2)])
        pltpu.sync_copy(
            o0.at[pl.ds(0, _TAIL // 2)],
            out_hbm.at[pl.ds(_NPANEL * 64, _TAIL // 2)])


@functools.partial(
    pl.kernel,
    mesh=_mesh,
    out_type=jax.ShapeDtypeStruct((_SEQ, 8, _NW, 8, 128), jnp.float32),
    scratch_types=(
        [pltpu.VMEM((_SEQ, 128), jnp.int32)]
        + [pltpu.VMEM((_SEQ, 128), jnp.int32)]
        + [pltpu.VMEM((128, 128), jnp.float32)] * 2
        + [pltpu.VMEM((8, 8, 128), jnp.float32)] * 2
        + [pltpu.SemaphoreType.DMA] * 4
    ),
    compiler_params=pltpu.CompilerParams(use_tc_tiling_on_sc=True, needs_layout_passes=False),
)
def _gather_kernel(xT_hbm, tab_hbm, out_hbm, idx_v, off_v,
                   g0, g1, o0, o1, gs0, gs1, ss0, ss1):
    wid = lax.axis_index("s") * _NC + lax.axis_index("c")
    gbufs, obufs = (g0, g1), (o0, o1)
    gsems, ssems = (gs0, gs1), (ss0, ss1)

    pltpu.sync_copy(xT_hbm.at[:, pl.ds(wid * 128, 128)], idx_v)

    def prep(s, carry):
        for g in range(8):
            ix = idx_v[s, pl.ds(16 * g, 16)]
            off_v[s, pl.ds(16 * g, 16)] = (ix & 1) << 6
            idx_v[s, pl.ds(16 * g, 16)] = lax.shift_right_logical(ix, 1)
        return carry
    lax.fori_loop(0, _SEQ, prep, 0, unroll=2)

    def fire_in(s, b):
        pltpu.async_copy(tab_hbm.at[idx_v.at[s]], gbufs[b], gsems[b])

    def wait_in(s, b):
        pltpu.make_async_copy(tab_hbm.at[idx_v.at[s]], gbufs[b], gsems[b]).wait()

    def fire_out(s, b):
        pltpu.async_copy(obufs[b], out_hbm.at[s, :, wid], ssems[b])

    def wait_out(s, b):
        pltpu.make_async_copy(obufs[b], out_hbm.at[s, :, wid], ssems[b]).wait()

    bi_idx = [lax.iota(jnp.int32, 16) + 16 * g for g in range(8)]

    def permute(s, g, o):
        offs = [off_v[s, pl.ds(16 * grp, 16)] for grp in range(8)]

        @plsc.parallel_loop(0, _D, unroll=4)
        def _(d):
            d0 = lax.shift_right_logical(d, 3)
            di = d & 7
            for grp in range(8):
                v = plsc.load_gather(g, [bi_idx[grp], offs[grp] + d])
                o[d0, di, pl.ds(16 * grp, 16)] = v * _SCALE

    fire_in(0, 0)

    def loop(t, carry):
        for b in range(2):
            s = 2 * t + b
            ns = s + 1

            @pl.when(ns < _SEQ)
            def _():
                fire_in(ns, 1 - b)
            wait_in(s, b)

            @pl.when(s >= 2)
            def _():
                wait_out(s - 2, b)
            permute(s, gbufs[b], obufs[b])
            fire_out(s, b)
        return carry

    lax.fori_loop(0, _SEQ // 2, loop, 0)
    wait_out(_SEQ - 2, 0)
    wait_out(_SEQ - 1, 1)


def kernel(x, input_embedding_table):
    tail = input_embedding_table[_NPANEL * 128:].reshape(_TAIL // 2, 128)
    tab2 = _transpose_kernel(input_embedding_table.T, tail)
    out5 = _gather_kernel(x.T, tab2)
    return out5.transpose(2, 4, 0, 1, 3).reshape(_BATCH, _SEQ, _D)
